# baseline (speedup 1.0000x reference)
import jax
import jax.numpy as jnp
from jax.experimental import pallas as pl

_BM = 512


def _router_block(x_ref, w_ref, o_ref):
    x = x_ref[...]
    w = w_ref[...]
    logits_t = jax.lax.dot_general(
        w, x,
        dimension_numbers=(((1,), (1,)), ((), ())),
        preferred_element_type=jnp.float32,
    )
    m = jnp.max(logits_t, axis=0, keepdims=True)
    e = jnp.exp(logits_t - m)
    o_ref[...] = e / jnp.sum(e, axis=0, keepdims=True)


def kernel(inputs, W):
    M, K = inputs.shape
    E = W.shape[0]
    grid = (M // _BM,)
    probs_t = pl.pallas_call(
        _router_block,
        grid=grid,
        in_specs=[
            pl.BlockSpec((_BM, K), lambda i: (i, 0)),
            pl.BlockSpec((E, K), lambda i: (0, 0)),
        ],
        out_specs=pl.BlockSpec((E, _BM), lambda i: (0, i)),
        out_shape=jax.ShapeDtypeStruct((E, M), jnp.float32),
    )(inputs, W)
    return probs_t.T


# speedup vs baseline: 1.1464x; 1.1464x over previous
import jax
import jax.numpy as jnp
from jax.experimental import pallas as pl

_BM = 2048


def _router_block(x_ref, w_ref, o_ref):
    x = x_ref[...]
    w = w_ref[...]
    logits_t = jax.lax.dot_general(
        w, x,
        dimension_numbers=(((1,), (1,)), ((), ())),
        preferred_element_type=jnp.float32,
    )
    m = jnp.max(logits_t, axis=0, keepdims=True)
    e = jnp.exp(logits_t - m)
    o_ref[...] = e / jnp.sum(e, axis=0, keepdims=True)


def kernel(inputs, W):
    M, K = inputs.shape
    E = W.shape[0]
    grid = (M // _BM,)
    probs_t = pl.pallas_call(
        _router_block,
        grid=grid,
        in_specs=[
            pl.BlockSpec((_BM, K), lambda i: (i, 0)),
            pl.BlockSpec((E, K), lambda i: (0, 0)),
        ],
        out_specs=pl.BlockSpec((E, _BM), lambda i: (0, i)),
        out_shape=jax.ShapeDtypeStruct((E, M), jnp.float32),
    )(inputs, W)
    return probs_t.T


# speedup vs baseline: 1.1651x; 1.0163x over previous
import jax
import jax.numpy as jnp
from jax.experimental import pallas as pl

_BM = 1024


def _router_block(x_ref, w_ref, o_ref):
    x = x_ref[...].astype(jnp.bfloat16)
    w = w_ref[...].astype(jnp.bfloat16)
    logits_t = jax.lax.dot_general(
        w, x,
        dimension_numbers=(((1,), (1,)), ((), ())),
        preferred_element_type=jnp.float32,
    )
    m = jnp.max(logits_t, axis=0, keepdims=True)
    e = jnp.exp(logits_t - m)
    o_ref[...] = e / jnp.sum(e, axis=0, keepdims=True)


def kernel(inputs, W):
    M, K = inputs.shape
    E = W.shape[0]
    grid = (M // _BM,)
    probs_t = pl.pallas_call(
        _router_block,
        grid=grid,
        in_specs=[
            pl.BlockSpec((_BM, K), lambda i: (i, 0)),
            pl.BlockSpec((E, K), lambda i: (0, 0)),
        ],
        out_specs=pl.BlockSpec((E, _BM), lambda i: (0, i)),
        out_shape=jax.ShapeDtypeStruct((E, M), jnp.float32),
    )(inputs, W)
    return probs_t.T
